# Initial kernel scaffold; baseline (speedup 1.0000x reference)
import jax
import jax.numpy as jnp
from jax.experimental import pallas as pl


def kernel(positions, cos_cached, sin_cached):
    raise NotImplementedError("write your pallas kernel here")


# speedup vs baseline: 7.0121x; 7.0121x over previous
import functools

import jax
import jax.numpy as jnp
from jax import lax
from jax.experimental import pallas as pl
from jax.experimental.pallas import tpu as pltpu
from jax.experimental.pallas import tpu_sc as plsc

DIM = 128
NC, NS = 2, 16
NW = NC * NS
CH = 128


def _make_gather(total):
    n_rows = total // CH
    rows_per_w = n_rows // NW
    mesh = plsc.VectorSubcoreMesh(core_axis_name="c", subcore_axis_name="s")

    @functools.partial(
        pl.kernel,
        out_type=(
            jax.ShapeDtypeStruct((total, DIM), jnp.float32),
            jax.ShapeDtypeStruct((total, DIM), jnp.float32),
        ),
        mesh=mesh,
        scratch_types=[
            pltpu.VMEM((rows_per_w, CH), jnp.int32),
            pltpu.VMEM((CH, DIM), jnp.float32),
            pltpu.VMEM((CH, DIM), jnp.float32),
            pltpu.SemaphoreType.DMA,
            pltpu.SemaphoreType.DMA,
        ],
    )
    def k(pos_hbm, cos_hbm, sin_hbm, cos_out, sin_out,
          idx_v, cos_rows, sin_rows, csem, ssem):
        wid = lax.axis_index("s") * NC + lax.axis_index("c")
        row0 = wid * rows_per_w
        pltpu.sync_copy(pos_hbm.at[pl.ds(row0, rows_per_w)], idx_v)
        for j in range(rows_per_w):
            idx_row = idx_v.at[j]
            cop = pltpu.async_copy(cos_hbm.at[idx_row], cos_rows, csem)
            sop = pltpu.async_copy(sin_hbm.at[idx_row], sin_rows, ssem)
            cop.wait()
            sop.wait()
            base = (row0 + j) * CH
            pltpu.sync_copy(cos_rows, cos_out.at[pl.ds(base, CH)])
            pltpu.sync_copy(sin_rows, sin_out.at[pl.ds(base, CH)])

    return k


def kernel(positions, cos_cached, sin_cached):
    b, s = positions.shape
    total = b * s
    pos = positions.reshape(total // CH, CH).astype(jnp.int32)
    cos_flat, sin_flat = _make_gather(total)(pos, cos_cached, sin_cached)
    return (cos_flat.reshape(b, s, DIM), sin_flat.reshape(b, s, DIM))


# speedup vs baseline: 8.2281x; 1.1734x over previous
import functools

import jax
import jax.numpy as jnp
from jax import lax
from jax.experimental import pallas as pl
from jax.experimental.pallas import tpu as pltpu
from jax.experimental.pallas import tpu_sc as plsc

DIM = 128
NC, NS = 2, 16
NW = NC * NS
CH = 128


NBUF = 3


def _make_gather(total):
    n_rows = total // CH
    rows_per_w = n_rows // NW
    mesh = plsc.VectorSubcoreMesh(core_axis_name="c", subcore_axis_name="s")

    @functools.partial(
        pl.kernel,
        out_type=(
            jax.ShapeDtypeStruct((total, DIM), jnp.float32),
            jax.ShapeDtypeStruct((total, DIM), jnp.float32),
        ),
        mesh=mesh,
        scratch_types=[
            pltpu.VMEM((rows_per_w, CH), jnp.int32),
            pltpu.VMEM((NBUF, CH, DIM), jnp.float32),
            pltpu.VMEM((NBUF, CH, DIM), jnp.float32),
        ]
        + [pltpu.SemaphoreType.DMA] * (4 * NBUF),
    )
    def k(pos_hbm, cos_hbm, sin_hbm, cos_out, sin_out,
          idx_v, cos_rows, sin_rows, *sems):
        cg, sg = sems[:NBUF], sems[NBUF : 2 * NBUF]
        cw, sw = sems[2 * NBUF : 3 * NBUF], sems[3 * NBUF :]
        wid = lax.axis_index("s") * NC + lax.axis_index("c")
        row0 = wid * rows_per_w
        pltpu.sync_copy(pos_hbm.at[pl.ds(row0, rows_per_w)], idx_v)
        gops = [None] * NBUF
        wops = [None] * NBUF
        for j in range(rows_per_w + 1):
            b = j % NBUF
            if j < rows_per_w:
                if wops[b] is not None:
                    wops[b][0].wait()
                    wops[b][1].wait()
                gops[b] = (
                    pltpu.async_copy(cos_hbm.at[idx_v.at[j]], cos_rows.at[b], cg[b]),
                    pltpu.async_copy(sin_hbm.at[idx_v.at[j]], sin_rows.at[b], sg[b]),
                )
            if j >= 1:
                pb = (j - 1) % NBUF
                gops[pb][0].wait()
                gops[pb][1].wait()
                base = (row0 + j - 1) * CH
                wops[pb] = (
                    pltpu.async_copy(cos_rows.at[pb], cos_out.at[pl.ds(base, CH)], cw[pb]),
                    pltpu.async_copy(sin_rows.at[pb], sin_out.at[pl.ds(base, CH)], sw[pb]),
                )
        for b in range(NBUF):
            wops[b][0].wait()
            wops[b][1].wait()

    return k


def kernel(positions, cos_cached, sin_cached):
    b, s = positions.shape
    total = b * s
    pos = positions.reshape(total // CH, CH).astype(jnp.int32)
    cos_flat, sin_flat = _make_gather(total)(pos, cos_cached, sin_cached)
    return (cos_flat.reshape(b, s, DIM), sin_flat.reshape(b, s, DIM))


# speedup vs baseline: 9.9164x; 1.2052x over previous
import functools

import jax
import jax.numpy as jnp
from jax import lax
from jax.experimental import pallas as pl
from jax.experimental.pallas import tpu as pltpu
from jax.experimental.pallas import tpu_sc as plsc

DIM = 128
NC, NS = 2, 16
NW = NC * NS
CH = 128


HALF = DIM // 2
NBUF = 3


def _make_gather(total):
    n_rows = total // CH
    rows_per_w = n_rows // NW
    mesh = plsc.VectorSubcoreMesh(core_axis_name="c", subcore_axis_name="s")

    @functools.partial(
        pl.kernel,
        out_type=(
            jax.ShapeDtypeStruct((total, 2, HALF), jnp.float32),
            jax.ShapeDtypeStruct((total, 2, HALF), jnp.float32),
        ),
        mesh=mesh,
        compiler_params=pltpu.CompilerParams(use_tc_tiling_on_sc=False),
        scratch_types=[
            pltpu.VMEM((rows_per_w, CH), jnp.int32),
            pltpu.VMEM((NBUF, CH, HALF), jnp.float32),
            pltpu.VMEM((NBUF, CH, HALF), jnp.float32),
        ]
        + [pltpu.SemaphoreType.DMA] * (6 * NBUF),
    )
    def k(pos_hbm, cos_hbm, sin_hbm, cos_out, sin_out,
          idx_v, cos_rows, sin_rows, *sems):
        cg, sg = sems[:NBUF], sems[NBUF : 2 * NBUF]
        wsems = sems[2 * NBUF :]
        wid = lax.axis_index("s") * NC + lax.axis_index("c")
        row0 = wid * rows_per_w
        pltpu.sync_copy(pos_hbm.at[pl.ds(row0, rows_per_w)], idx_v)
        for j in range(rows_per_w):
            for i in range(CH // 16):
                sl = (j, pl.ds(i * 16, 16))
                idx_v[sl] = idx_v[sl] * 2
        gops = [None] * NBUF
        wops = [None] * NBUF
        for j in range(rows_per_w + 1):
            b = j % NBUF
            if j < rows_per_w:
                if wops[b] is not None:
                    for w in wops[b]:
                        w.wait()
                gops[b] = (
                    pltpu.async_copy(cos_hbm.at[idx_v.at[j]], cos_rows.at[b], cg[b]),
                    pltpu.async_copy(sin_hbm.at[idx_v.at[j]], sin_rows.at[b], sg[b]),
                )
            if j >= 1:
                pb = (j - 1) % NBUF
                gops[pb][0].wait()
                gops[pb][1].wait()
                base = (row0 + j - 1) * CH
                wops[pb] = tuple(
                    pltpu.async_copy(
                        rows.at[pb],
                        out.at[pl.ds(base, CH), h],
                        wsems[pb * 4 + wi],
                    )
                    for wi, (rows, out, h) in enumerate(
                        [(cos_rows, cos_out, 0), (cos_rows, cos_out, 1),
                         (sin_rows, sin_out, 0), (sin_rows, sin_out, 1)]
                    )
                )
        for b in range(NBUF):
            for w in wops[b]:
                w.wait()

    return k


def kernel(positions, cos_cached, sin_cached):
    b, s = positions.shape
    total = b * s
    pos = positions.reshape(total // CH, CH).astype(jnp.int32)
    cos_half = cos_cached.reshape(2 * cos_cached.shape[0], HALF)
    sin_half = sin_cached.reshape(2 * sin_cached.shape[0], HALF)
    cos_flat, sin_flat = _make_gather(total)(pos, cos_half, sin_half)
    return (cos_flat.reshape(b, s, DIM), sin_flat.reshape(b, s, DIM))


# speedup vs baseline: 9.9588x; 1.0043x over previous
import functools

import jax
import jax.numpy as jnp
from jax import lax
from jax.experimental import pallas as pl
from jax.experimental.pallas import tpu as pltpu
from jax.experimental.pallas import tpu_sc as plsc

DIM = 128
NC, NS = 2, 16
NW = NC * NS
CH = 128


HALF = DIM // 2
NBUF = 4


def _make_gather(total):
    n_rows = total // CH
    rows_per_w = n_rows // NW
    mesh = plsc.VectorSubcoreMesh(core_axis_name="c", subcore_axis_name="s")

    @functools.partial(
        pl.kernel,
        out_type=(
            jax.ShapeDtypeStruct((total, 2, HALF), jnp.float32),
            jax.ShapeDtypeStruct((total, 2, HALF), jnp.float32),
        ),
        mesh=mesh,
        compiler_params=pltpu.CompilerParams(use_tc_tiling_on_sc=False),
        scratch_types=[
            pltpu.VMEM((rows_per_w, CH), jnp.int32),
            pltpu.VMEM((NBUF, CH, HALF), jnp.float32),
            pltpu.VMEM((NBUF, CH, HALF), jnp.float32),
        ]
        + [pltpu.SemaphoreType.DMA] * (6 * NBUF),
    )
    def k(pos_hbm, cos_hbm, sin_hbm, cos_out, sin_out,
          idx_v, cos_rows, sin_rows, *sems):
        cg, sg = sems[:NBUF], sems[NBUF : 2 * NBUF]
        wsems = sems[2 * NBUF :]
        wid = lax.axis_index("s") * NC + lax.axis_index("c")
        row0 = wid * rows_per_w
        pltpu.sync_copy(pos_hbm.at[pl.ds(row0, rows_per_w)], idx_v)
        gops = [None] * NBUF
        wops = [None] * NBUF
        for j in range(rows_per_w + 1):
            b = j % NBUF
            if j < rows_per_w:
                for i in range(CH // 16):
                    sl = (j, pl.ds(i * 16, 16))
                    idx_v[sl] = idx_v[sl] * 2
                if wops[b] is not None:
                    for w in wops[b]:
                        w.wait()
                gops[b] = (
                    pltpu.async_copy(cos_hbm.at[idx_v.at[j]], cos_rows.at[b], cg[b]),
                    pltpu.async_copy(sin_hbm.at[idx_v.at[j]], sin_rows.at[b], sg[b]),
                )
            if j >= 1:
                pb = (j - 1) % NBUF
                gops[pb][0].wait()
                gops[pb][1].wait()
                base = (row0 + j - 1) * CH
                wops[pb] = tuple(
                    pltpu.async_copy(
                        rows.at[pb],
                        out.at[pl.ds(base, CH), h],
                        wsems[pb * 4 + wi],
                    )
                    for wi, (rows, out, h) in enumerate(
                        [(cos_rows, cos_out, 0), (cos_rows, cos_out, 1),
                         (sin_rows, sin_out, 0), (sin_rows, sin_out, 1)]
                    )
                )
        for b in range(NBUF):
            for w in wops[b]:
                w.wait()

    return k


def kernel(positions, cos_cached, sin_cached):
    b, s = positions.shape
    total = b * s
    pos = positions.reshape(total // CH, CH).astype(jnp.int32)
    cos_half = cos_cached.reshape(2 * cos_cached.shape[0], HALF)
    sin_half = sin_cached.reshape(2 * sin_cached.shape[0], HALF)
    cos_flat, sin_flat = _make_gather(total)(pos, cos_half, sin_half)
    return (cos_flat.reshape(b, s, DIM), sin_flat.reshape(b, s, DIM))
